# baseline (speedup 1.0000x reference)
import functools

import numpy as _numpy

import jax
import jax.numpy as jnp
from jax import lax
from jax.experimental import pallas as pl
from jax.experimental.pallas import tpu as pltpu
from jax.experimental.pallas import tpu_sc as plsc

S = 2048
D = 1024
E = 8
C = 2 * S // E
SLOTS = E * C
NC, NS, L = 2, 16, 16
NW = NC * NS
SLOTS_PER_W = SLOTS // NW
TOK_PER_W = S // NW


def _gating_body(x_ref, wg_ref, gum_ref,
                 dst1_ref, dst2_ref, g1_ref, g2_ref, laux_ref, xlin_ref,
                 m1_ref, m2_ref, l1_ref, l2_ref):
    x = x_ref[...]
    xlin_ref[...] = x
    wg = wg_ref[...]
    logits = jnp.dot(x, wg, preferred_element_type=jnp.float32)

    mx = jnp.max(logits, axis=1, keepdims=True)
    ex = jnp.exp(logits - mx)
    gates = ex / jnp.sum(ex, axis=1, keepdims=True)

    col = lax.broadcasted_iota(jnp.int32, (S, E), 1)

    gmax = jnp.max(gates, axis=1, keepdims=True)
    e1 = jnp.min(jnp.where(gates == gmax, col, E), axis=1, keepdims=True)
    mask1 = (col == e1).astype(jnp.float32)

    lw = logits + gum_ref[...]
    neg_inf = jnp.float32(-jnp.inf)
    lw2 = jnp.where(mask1 > 0, neg_inf, lw)
    l2max = jnp.max(lw2, axis=1, keepdims=True)
    e2 = jnp.min(jnp.where(lw2 == l2max, col, E), axis=1, keepdims=True)
    mask2 = (col == e2).astype(jnp.float32)

    m1_ref[...] = mask1
    m2_ref[...] = mask2
    tri = (lax.broadcasted_iota(jnp.int32, (128, 128), 0)
           >= lax.broadcasted_iota(jnp.int32, (128, 128), 1)
           ).astype(jnp.float32)

    def body(i, carry):
        c1, c2 = carry
        mb1 = m1_ref[pl.ds(i * 128, 128), :]
        mb2 = m2_ref[pl.ds(i * 128, 128), :]
        l1_ref[pl.ds(i * 128, 128), :] = (
            jnp.dot(tri, mb1, preferred_element_type=jnp.float32) + c1 - 1.0)
        l2_ref[pl.ds(i * 128, 128), :] = (
            jnp.dot(tri, mb2, preferred_element_type=jnp.float32) + c2 - 1.0)
        return (c1 + jnp.sum(mb1, axis=0, keepdims=True),
                c2 + jnp.sum(mb2, axis=0, keepdims=True))

    zero8 = jnp.zeros((1, E), jnp.float32)
    c1f, _ = lax.fori_loop(0, S // 128, body, (zero8, zero8))

    loc1 = l1_ref[...]
    loc2 = l2_ref[...] + c1f

    loc1_s = jnp.sum(loc1 * mask1, axis=1, keepdims=True)
    loc2_s = jnp.sum(loc2 * mask2, axis=1, keepdims=True)
    keep1 = (loc1_s < C).astype(jnp.float32)
    keep2 = (loc2_s < C).astype(jnp.float32)

    g1_raw = jnp.sum(gates * mask1, axis=1, keepdims=True) * keep1
    g2_raw = jnp.sum(gates * mask2, axis=1, keepdims=True) * keep2
    denom = jnp.maximum(g1_raw + g2_raw, jnp.finfo(jnp.float32).eps)
    g1_ref[...] = (g1_raw / denom).reshape(S)
    g2_ref[...] = (g2_raw / denom).reshape(S)

    dst1 = e1 * C + loc1_s.astype(jnp.int32)
    dst2 = e2 * C + loc2_s.astype(jnp.int32)
    dst1_ref[...] = jnp.where(keep1 > 0, dst1, -1).reshape(S)
    dst2_ref[...] = jnp.where(keep2 > 0, dst2, -1).reshape(S)

    me_ce = (jnp.sum(gates, axis=0, keepdims=True)
             * jnp.sum(mask1, axis=0, keepdims=True))
    laux_ref[...] = (jnp.sum(me_ce, axis=1, keepdims=True)
                     * (jnp.float32(E * E) / jnp.float32(S * S * E)))


_gating_call = pl.pallas_call(
    _gating_body,
    out_shape=(
        jax.ShapeDtypeStruct((S,), jnp.int32),
        jax.ShapeDtypeStruct((S,), jnp.int32),
        jax.ShapeDtypeStruct((S,), jnp.float32),
        jax.ShapeDtypeStruct((S,), jnp.float32),
        jax.ShapeDtypeStruct((1, 1), jnp.float32),
        jax.ShapeDtypeStruct((S, D), jnp.float32),
    ),
    scratch_shapes=[pltpu.VMEM((S, E), jnp.float32) for _ in range(4)],
)


_SC_PARAMS = pltpu.CompilerParams(needs_layout_passes=False)


@functools.cache
def _get_dispatch():
  mesh = plsc.VectorSubcoreMesh(core_axis_name="c", subcore_axis_name="s")

  @functools.partial(
      pl.kernel,
      mesh=mesh,
      compiler_params=_SC_PARAMS,
      out_type=jax.ShapeDtypeStruct((SLOTS, D), jnp.float32),
      scratch_types=[
          pltpu.VMEM((S,), jnp.int32),
          pltpu.VMEM((SLOTS_PER_W,), jnp.int32),
          pltpu.VMEM((32, D), jnp.float32),
          pltpu.VMEM((32, D), jnp.float32),
          pltpu.SemaphoreType.DMA,
          pltpu.SemaphoreType.DMA,
      ],
  )
  def _dispatch(x_hbm, dst1_hbm, dst2_hbm, disp_hbm,
                dst_v, map_v, rows_a, rows_b, sem_a, sem_b):
    wid = lax.axis_index("s") * NC + lax.axis_index("c")
    base = wid * SLOTS_PER_W

    for i in range(SLOTS_PER_W // L):
        map_v[pl.ds(i * L, L)] = jnp.zeros((L,), jnp.int32)

    def scatter_from(dst_hbm):
        pltpu.sync_copy(dst_hbm, dst_v)

        @pl.loop(0, S // L // 16)
        def _scat(o):
            for k in range(16):
                i = o * 16 + k
                d = dst_v[pl.ds(i * L, L)]
                t = i * L + lax.iota(jnp.int32, L)
                di = d - base
                msk = (di >= 0) & (di < SLOTS_PER_W)
                di = jnp.where(msk, di, 0)
                plsc.store_scatter(map_v, [di], t, mask=msk)

    scatter_from(dst1_hbm)
    scatter_from(dst2_hbm)

    nch = SLOTS_PER_W // 32

    def gather(j, buf, sem):
        return pltpu.async_copy(
            x_hbm.at[map_v.at[pl.ds(j * 32, 32)]], buf, sem)

    bufs = (rows_a, rows_b)
    sems = (sem_a, sem_b)
    gather(0, rows_a, sem_a).wait()
    for j in range(nch):
        nxt = None
        if j + 1 < nch:
            nxt = gather(j + 1, bufs[(j + 1) % 2], sems[(j + 1) % 2])
        pltpu.sync_copy(bufs[j % 2], disp_hbm.at[pl.ds(base + j * 32, 32)])
        if nxt is not None:
            nxt.wait()

  return _dispatch


def _ffn_body(d_ref, w1_ref, b1_ref, w2_ref, b2_ref, o_ref):
    d = d_ref[...].astype(jnp.bfloat16)
    w1 = w1_ref[0].astype(jnp.bfloat16)
    h = jnp.dot(d, w1, preferred_element_type=jnp.float32)
    h = jnp.maximum(h + b1_ref[0], 0.0).astype(jnp.bfloat16)
    w2 = w2_ref[0].astype(jnp.bfloat16)
    o = jnp.dot(h, w2, preferred_element_type=jnp.float32)
    o_ref[...] = o + b2_ref[0]


_ffn_call = pl.pallas_call(
    _ffn_body,
    grid=(E,),
    in_specs=[
        pl.BlockSpec((C, D), lambda e: (e, 0)),
        pl.BlockSpec((1, D, D), lambda e: (e, 0, 0)),
        pl.BlockSpec((1, 1, D), lambda e: (e, 0, 0)),
        pl.BlockSpec((1, D, D), lambda e: (e, 0, 0)),
        pl.BlockSpec((1, 1, D), lambda e: (e, 0, 0)),
    ],
    out_specs=pl.BlockSpec((C, D), lambda e: (e, 0)),
    out_shape=jax.ShapeDtypeStruct((SLOTS, D), jnp.float32),
)


def _emit_body(i_ref, o_ref):
    o_ref[...] = i_ref[...]


_emit_call = pl.pallas_call(
    _emit_body,
    grid=(8,),
    in_specs=[pl.BlockSpec((S // 8, D), lambda i: (i, 0))],
    out_specs=pl.BlockSpec((S // 8, D), lambda i: (i, 0)),
    out_shape=jax.ShapeDtypeStruct((S, D), jnp.float32),
)


@functools.cache
def _get_combine():
  mesh = plsc.VectorSubcoreMesh(core_axis_name="c", subcore_axis_name="s")

  @functools.partial(
      pl.kernel,
      mesh=mesh,
      compiler_params=_SC_PARAMS,
      out_type=jax.ShapeDtypeStruct((S, D), jnp.float32),
      scratch_types=[
          pltpu.VMEM((TOK_PER_W,), jnp.int32),
          pltpu.VMEM((TOK_PER_W,), jnp.int32),
          pltpu.VMEM((TOK_PER_W + L,), jnp.float32),
          pltpu.VMEM((TOK_PER_W + L,), jnp.float32),
          pltpu.VMEM((16, D), jnp.float32),
          pltpu.VMEM((16, D), jnp.float32),
          pltpu.VMEM((16, D), jnp.float32),
          pltpu.VMEM((16, D), jnp.float32),
          pltpu.SemaphoreType.DMA,
          pltpu.SemaphoreType.DMA,
          pltpu.SemaphoreType.DMA,
          pltpu.SemaphoreType.DMA,
      ],
  )
  def _combine(eo_hbm, dst1_hbm, dst2_hbm, g1_hbm, g2_hbm, out_hbm,
               idx1_v, idx2_v, g1_v, g2_v,
               r1a, r2a, r1b, r2b, s1a, s2a, s1b, s2b):
    wid = lax.axis_index("s") * NC + lax.axis_index("c")
    tbase = wid * TOK_PER_W
    pltpu.sync_copy(dst1_hbm.at[pl.ds(tbase, TOK_PER_W)], idx1_v)
    pltpu.sync_copy(dst2_hbm.at[pl.ds(tbase, TOK_PER_W)], idx2_v)
    pltpu.sync_copy(g1_hbm.at[pl.ds(tbase, TOK_PER_W)],
                    g1_v.at[pl.ds(0, TOK_PER_W)])
    pltpu.sync_copy(g2_hbm.at[pl.ds(tbase, TOK_PER_W)],
                    g2_v.at[pl.ds(0, TOK_PER_W)])

    @pl.loop(0, TOK_PER_W // L)
    def _clamp(i):
        d1 = idx1_v[pl.ds(i * L, L)]
        idx1_v[pl.ds(i * L, L)] = jnp.maximum(d1, 0)
        d2 = idx2_v[pl.ds(i * L, L)]
        idx2_v[pl.ds(i * L, L)] = jnp.maximum(d2, 0)

    CH = 16
    nch = TOK_PER_W // CH
    bufs = ((r1a, r2a, s1a, s2a), (r1b, r2b, s1b, s2b))

    def gathers(j, bset):
        b1, b2, s1, s2 = bset
        c1 = pltpu.async_copy(eo_hbm.at[idx1_v.at[pl.ds(j * CH, CH)]], b1, s1)
        c2 = pltpu.async_copy(eo_hbm.at[idx2_v.at[pl.ds(j * CH, CH)]], b2, s2)
        return c1, c2

    p1, p2 = gathers(0, bufs[0])
    p1.wait()
    p2.wait()
    for j in range(nch):
        if j + 1 < nch:
            n1, n2 = gathers(j + 1, bufs[(j + 1) % 2])
        b1, b2, _, _ = bufs[j % 2]

        @pl.loop(0, CH)
        def _row(r, _j=j, _b1=b1, _b2=b2):
            s1 = g1_v[pl.ds(_j * CH + r, L)][0]
            s2 = g2_v[pl.ds(_j * CH + r, L)][0]
            for c in range(0, D, L):
                _b1[r, pl.ds(c, L)] = (s1 * _b1[r, pl.ds(c, L)]
                                       + s2 * _b2[r, pl.ds(c, L)])

        pltpu.sync_copy(b1, out_hbm.at[pl.ds(tbase + j * CH, CH)])
        if j + 1 < nch:
            n1.wait()
            n2.wait()

  return _combine


_GUMBEL = _numpy.asarray(
    jax.random.gumbel(jax.random.key(1), (S, E), dtype=jnp.float32))


def kernel(x, wg, w1, b1, w2, b2):
    x2d = x.reshape(S, D)
    gumbel = jnp.asarray(_GUMBEL)

    dst1, dst2, g1, g2, laux, xlin = _gating_call(x2d, wg, gumbel)

    disp = _get_dispatch()(xlin, dst1, dst2)

    eo = _ffn_call(disp,
                   w1, b1.reshape(E, 1, D),
                   w2, b2.reshape(E, 1, D))

    out = _emit_call(_get_combine()(eo, dst1, dst2, g1, g2))

    return out.reshape(x.shape), laux.reshape(())


# speedup vs baseline: 1.0529x; 1.0529x over previous
import functools

import numpy as _numpy

import jax
import jax.numpy as jnp
from jax import lax
from jax.experimental import pallas as pl
from jax.experimental.pallas import tpu as pltpu
from jax.experimental.pallas import tpu_sc as plsc

S = 2048
D = 1024
E = 8
C = 2 * S // E
SLOTS = E * C
NC, NS, L = 2, 16, 16
NW = NC * NS
SLOTS_PER_W = SLOTS // NW
TOK_PER_W = S // NW


def _gating_body(x_ref, wg_ref, gum_ref,
                 dst1_ref, dst2_ref, g1_ref, g2_ref, laux_ref,
                 m1_ref, m2_ref, l1_ref, l2_ref):
    x = x_ref[...]
    wg = wg_ref[...]
    logits = jnp.dot(x, wg, preferred_element_type=jnp.float32)

    mx = jnp.max(logits, axis=1, keepdims=True)
    ex = jnp.exp(logits - mx)
    gates = ex / jnp.sum(ex, axis=1, keepdims=True)

    col = lax.broadcasted_iota(jnp.int32, (S, E), 1)

    gmax = jnp.max(gates, axis=1, keepdims=True)
    e1 = jnp.min(jnp.where(gates == gmax, col, E), axis=1, keepdims=True)
    mask1 = (col == e1).astype(jnp.float32)

    lw = logits + gum_ref[...]
    neg_inf = jnp.float32(-jnp.inf)
    lw2 = jnp.where(mask1 > 0, neg_inf, lw)
    l2max = jnp.max(lw2, axis=1, keepdims=True)
    e2 = jnp.min(jnp.where(lw2 == l2max, col, E), axis=1, keepdims=True)
    mask2 = (col == e2).astype(jnp.float32)

    m1_ref[...] = mask1
    m2_ref[...] = mask2
    TB = 512
    tri = (lax.broadcasted_iota(jnp.int32, (TB, TB), 0)
           >= lax.broadcasted_iota(jnp.int32, (TB, TB), 1)
           ).astype(jnp.float32)

    def body(i, carry):
        c1, c2 = carry
        mb1 = m1_ref[pl.ds(i * TB, TB), :]
        mb2 = m2_ref[pl.ds(i * TB, TB), :]
        l1_ref[pl.ds(i * TB, TB), :] = (
            jnp.dot(tri, mb1, preferred_element_type=jnp.float32) + c1 - 1.0)
        l2_ref[pl.ds(i * TB, TB), :] = (
            jnp.dot(tri, mb2, preferred_element_type=jnp.float32) + c2 - 1.0)
        return (c1 + jnp.sum(mb1, axis=0, keepdims=True),
                c2 + jnp.sum(mb2, axis=0, keepdims=True))

    zero8 = jnp.zeros((1, E), jnp.float32)
    c1f, _ = lax.fori_loop(0, S // TB, body, (zero8, zero8))

    loc1 = l1_ref[...]
    loc2 = l2_ref[...] + c1f

    loc1_s = jnp.sum(loc1 * mask1, axis=1, keepdims=True)
    loc2_s = jnp.sum(loc2 * mask2, axis=1, keepdims=True)
    keep1 = (loc1_s < C).astype(jnp.float32)
    keep2 = (loc2_s < C).astype(jnp.float32)

    g1_raw = jnp.sum(gates * mask1, axis=1, keepdims=True) * keep1
    g2_raw = jnp.sum(gates * mask2, axis=1, keepdims=True) * keep2
    denom = jnp.maximum(g1_raw + g2_raw, jnp.finfo(jnp.float32).eps)
    g1_ref[...] = (g1_raw / denom).reshape(S)
    g2_ref[...] = (g2_raw / denom).reshape(S)

    dst1 = e1 * C + loc1_s.astype(jnp.int32)
    dst2 = e2 * C + loc2_s.astype(jnp.int32)
    dst1_ref[...] = jnp.where(keep1 > 0, dst1, -1).reshape(S)
    dst2_ref[...] = jnp.where(keep2 > 0, dst2, -1).reshape(S)

    me_ce = (jnp.sum(gates, axis=0, keepdims=True)
             * jnp.sum(mask1, axis=0, keepdims=True))
    laux_ref[...] = (jnp.sum(me_ce, axis=1, keepdims=True)
                     * (jnp.float32(E * E) / jnp.float32(S * S * E)))


_gating_call = pl.pallas_call(
    _gating_body,
    out_shape=(
        jax.ShapeDtypeStruct((S,), jnp.int32),
        jax.ShapeDtypeStruct((S,), jnp.int32),
        jax.ShapeDtypeStruct((S,), jnp.float32),
        jax.ShapeDtypeStruct((S,), jnp.float32),
        jax.ShapeDtypeStruct((1, 1), jnp.float32),
    ),
    scratch_shapes=[pltpu.VMEM((S, E), jnp.float32) for _ in range(4)],
)


_SC_PARAMS = pltpu.CompilerParams(needs_layout_passes=False)


@functools.cache
def _get_dispatch():
  mesh = plsc.VectorSubcoreMesh(core_axis_name="c", subcore_axis_name="s")

  @functools.partial(
      pl.kernel,
      mesh=mesh,
      compiler_params=_SC_PARAMS,
      out_type=jax.ShapeDtypeStruct((SLOTS, D), jnp.float32),
      scratch_types=[
          pltpu.VMEM((S,), jnp.int32),
          pltpu.VMEM((SLOTS_PER_W,), jnp.int32),
          pltpu.VMEM((32, D), jnp.float32),
          pltpu.VMEM((32, D), jnp.float32),
          pltpu.SemaphoreType.DMA,
          pltpu.SemaphoreType.DMA,
      ],
  )
  def _dispatch(x_hbm, dst1_hbm, dst2_hbm, disp_hbm,
                dst_v, map_v, rows_a, rows_b, sem_a, sem_b):
    wid = lax.axis_index("s") * NC + lax.axis_index("c")
    base = wid * SLOTS_PER_W

    for i in range(SLOTS_PER_W // L):
        map_v[pl.ds(i * L, L)] = jnp.zeros((L,), jnp.int32)

    def scatter_from(dst_hbm):
        pltpu.sync_copy(dst_hbm, dst_v)

        @pl.loop(0, S // L // 16)
        def _scat(o):
            for k in range(16):
                i = o * 16 + k
                d = dst_v[pl.ds(i * L, L)]
                t = i * L + lax.iota(jnp.int32, L)
                di = d - base
                msk = (di >= 0) & (di < SLOTS_PER_W)
                di = jnp.where(msk, di, 0)
                plsc.store_scatter(map_v, [di], t, mask=msk)

    scatter_from(dst1_hbm)
    scatter_from(dst2_hbm)

    nch = SLOTS_PER_W // 32

    def gather(j, buf, sem):
        return pltpu.async_copy(
            x_hbm.at[map_v.at[pl.ds(j * 32, 32)]], buf, sem)

    bufs = (rows_a, rows_b)
    sems = (sem_a, sem_b)
    gather(0, rows_a, sem_a).wait()
    for j in range(nch):
        nxt = None
        if j + 1 < nch:
            nxt = gather(j + 1, bufs[(j + 1) % 2], sems[(j + 1) % 2])
        pltpu.sync_copy(bufs[j % 2], disp_hbm.at[pl.ds(base + j * 32, 32)])
        if nxt is not None:
            nxt.wait()

  return _dispatch


def _ffn_body(d_ref, w1_ref, b1_ref, w2_ref, b2_ref, o_ref):
    d = d_ref[...].astype(jnp.bfloat16)
    w1 = w1_ref[0].astype(jnp.bfloat16)
    h = jnp.dot(d, w1, preferred_element_type=jnp.float32)
    h = jnp.maximum(h + b1_ref[0], 0.0).astype(jnp.bfloat16)
    w2 = w2_ref[0].astype(jnp.bfloat16)
    o = jnp.dot(h, w2, preferred_element_type=jnp.float32)
    o_ref[...] = o + b2_ref[0]


_ffn_call = pl.pallas_call(
    _ffn_body,
    grid=(E,),
    in_specs=[
        pl.BlockSpec((C, D), lambda e: (e, 0)),
        pl.BlockSpec((1, D, D), lambda e: (e, 0, 0)),
        pl.BlockSpec((1, 1, D), lambda e: (e, 0, 0)),
        pl.BlockSpec((1, D, D), lambda e: (e, 0, 0)),
        pl.BlockSpec((1, 1, D), lambda e: (e, 0, 0)),
    ],
    out_specs=pl.BlockSpec((C, D), lambda e: (e, 0)),
    out_shape=jax.ShapeDtypeStruct((SLOTS, D), jnp.float32),
)


@functools.cache
def _get_combine():
  mesh = plsc.VectorSubcoreMesh(core_axis_name="c", subcore_axis_name="s")

  @functools.partial(
      pl.kernel,
      mesh=mesh,
      compiler_params=_SC_PARAMS,
      out_type=jax.ShapeDtypeStruct((S, D), jnp.float32),
      scratch_types=[
          pltpu.VMEM((TOK_PER_W,), jnp.int32),
          pltpu.VMEM((TOK_PER_W,), jnp.int32),
          pltpu.VMEM((TOK_PER_W + L,), jnp.float32),
          pltpu.VMEM((TOK_PER_W + L,), jnp.float32),
          pltpu.VMEM((16, D), jnp.float32),
          pltpu.VMEM((16, D), jnp.float32),
          pltpu.VMEM((16, D), jnp.float32),
          pltpu.VMEM((16, D), jnp.float32),
          pltpu.SemaphoreType.DMA,
          pltpu.SemaphoreType.DMA,
          pltpu.SemaphoreType.DMA,
          pltpu.SemaphoreType.DMA,
      ],
  )
  def _combine(eo_hbm, dst1_hbm, dst2_hbm, g1_hbm, g2_hbm, out_hbm,
               idx1_v, idx2_v, g1_v, g2_v,
               r1a, r2a, r1b, r2b, s1a, s2a, s1b, s2b):
    wid = lax.axis_index("s") * NC + lax.axis_index("c")
    tbase = wid * TOK_PER_W
    pltpu.sync_copy(dst1_hbm.at[pl.ds(tbase, TOK_PER_W)], idx1_v)
    pltpu.sync_copy(dst2_hbm.at[pl.ds(tbase, TOK_PER_W)], idx2_v)
    pltpu.sync_copy(g1_hbm.at[pl.ds(tbase, TOK_PER_W)],
                    g1_v.at[pl.ds(0, TOK_PER_W)])
    pltpu.sync_copy(g2_hbm.at[pl.ds(tbase, TOK_PER_W)],
                    g2_v.at[pl.ds(0, TOK_PER_W)])

    @pl.loop(0, TOK_PER_W // L)
    def _clamp(i):
        d1 = idx1_v[pl.ds(i * L, L)]
        idx1_v[pl.ds(i * L, L)] = jnp.maximum(d1, 0)
        d2 = idx2_v[pl.ds(i * L, L)]
        idx2_v[pl.ds(i * L, L)] = jnp.maximum(d2, 0)

    CH = 16
    nch = TOK_PER_W // CH
    bufs = ((r1a, r2a, s1a, s2a), (r1b, r2b, s1b, s2b))

    def gathers(j, bset):
        b1, b2, s1, s2 = bset
        c1 = pltpu.async_copy(eo_hbm.at[idx1_v.at[pl.ds(j * CH, CH)]], b1, s1)
        c2 = pltpu.async_copy(eo_hbm.at[idx2_v.at[pl.ds(j * CH, CH)]], b2, s2)
        return c1, c2

    p1, p2 = gathers(0, bufs[0])
    p1.wait()
    p2.wait()
    for j in range(nch):
        if j + 1 < nch:
            n1, n2 = gathers(j + 1, bufs[(j + 1) % 2])
        b1, b2, _, _ = bufs[j % 2]

        @pl.loop(0, CH)
        def _row(r, _j=j, _b1=b1, _b2=b2):
            s1 = g1_v[pl.ds(_j * CH + r, L)][0]
            s2 = g2_v[pl.ds(_j * CH + r, L)][0]
            for c in range(0, D, L):
                _b1[r, pl.ds(c, L)] = (s1 * _b1[r, pl.ds(c, L)]
                                       + s2 * _b2[r, pl.ds(c, L)])

        pltpu.sync_copy(b1, out_hbm.at[pl.ds(tbase + j * CH, CH)])
        if j + 1 < nch:
            n1.wait()
            n2.wait()

  return _combine


_GUMBEL = _numpy.asarray(
    jax.random.gumbel(jax.random.key(1), (S, E), dtype=jnp.float32))


def kernel(x, wg, w1, b1, w2, b2):
    x2d = x.reshape(S, D)
    gumbel = jnp.asarray(_GUMBEL)

    dst1, dst2, g1, g2, laux = _gating_call(x2d, wg, gumbel)

    disp = _get_dispatch()(x2d, dst1, dst2)

    eo = _ffn_call(disp,
                   w1, b1.reshape(E, 1, D),
                   w2, b2.reshape(E, 1, D))

    out = _get_combine()(eo, dst1, dst2, g1, g2)

    return out.reshape(x.shape), laux.reshape(())
